# baseline (speedup 1.0000x reference)
import functools

import jax
import jax.numpy as jnp
from jax.experimental import pallas as pl
from jax.experimental.pallas import tpu as pltpu

_NG = 64
_DIN = 128
_DHID = 256
_DEMB = 124
_NP = 4
_BLK = 4096
_NC = 2


def _fused_body(xref, bref, w1ref, b1ref, wg2ref, bg2ref, wn2ref,
                mout, sout, pout, mref, sref, pref, *, nblk_c, blk, n_real):
    c = pl.program_id(0)
    j = pl.program_id(1)
    i = c * nblk_c + j

    @pl.when(j == 0)
    def _init():
        mref[...] = jnp.full_like(mref, -1e30)
        sref[...] = jnp.zeros_like(sref)
        pref[...] = jnp.zeros_like(pref)

    rowi = jax.lax.broadcasted_iota(jnp.int32, (blk, 1), 0)
    valid = (rowi + i * blk) < n_real
    xb = jnp.where(valid, xref[...], 0.0).astype(jnp.bfloat16)
    h = jnp.dot(xb, w1ref[...], preferred_element_type=jnp.float32)
    hb = jnp.maximum(h + b1ref[...], 0.0).astype(jnp.bfloat16)
    hg = hb[:, : _NP * _DHID]
    g4 = jnp.dot(hg, wg2ref[...], preferred_element_type=jnp.float32)
    g4 = g4 + bg2ref[...]

    g4 = jnp.where(valid, g4, -1e30)

    mb = jnp.max(g4, axis=0, keepdims=True)
    mo = mref[...]
    mn = jnp.maximum(mo, mb)
    scale = jnp.exp(mo - mn)
    mref[...] = mn
    e4 = jnp.exp(g4 - mn)

    bb = bref[...].reshape(blk, 1)
    gid = jax.lax.broadcasted_iota(jnp.int32, (blk, _NG), 1).astype(jnp.float32)
    onehot = (bb == gid).astype(jnp.float32)
    snew = jax.lax.dot_general(onehot, e4, (((0,), (0,)), ((), ())),
                               preferred_element_type=jnp.float32)
    sref[...] = sref[...] * scale + snew

    for k in range(_NP):
        hk = hb[:, _NP * _DHID + k * _DHID: _NP * _DHID + (k + 1) * _DHID]
        ok = jnp.dot(hk, wn2ref[k], preferred_element_type=jnp.float32)
        wk = onehot * e4[:, k:k + 1]
        pk = jax.lax.dot_general(wk, ok, (((0,), (0,)), ((), ())),
                                 preferred_element_type=jnp.float32)
        pref[k] = pref[k] * scale[0, k] + pk

    @pl.when(j == nblk_c - 1)
    def _fin():
        mout[0] = mref[...]
        sout[0] = sref[...]
        pout[0] = pref[...]


def _pooled(x, batch, Wg1, bg1, Wg2, bg2, Wn1, bn1, Wn2, bn2, interpret=False):
    n = x.shape[0]
    blk = _BLK
    nblk = -(-n // blk)
    nblk_c = -(-nblk // _NC)
    bf = batch.astype(jnp.float32).reshape(1, n)

    w1 = jnp.concatenate(
        [jnp.transpose(Wg1, (1, 0, 2)).reshape(_DIN, _NP * _DHID),
         jnp.transpose(Wn1, (1, 0, 2)).reshape(_DIN, _NP * _DHID)],
        axis=1).astype(jnp.bfloat16)
    b1 = jnp.concatenate([bg1.reshape(1, -1), bn1.reshape(1, -1)], axis=1)
    ar = jnp.arange(_NP)
    wg2 = jnp.zeros((_NP, _DHID, _NP), jnp.float32)
    wg2 = wg2.at[ar, :, ar].set(Wg2[:, :, 0]).reshape(
        _NP * _DHID, _NP).astype(jnp.bfloat16)
    bg2r = bg2.reshape(1, _NP)

    def _rowmap(c, j):
        return (jnp.minimum(c * nblk_c + j, nblk - 1), 0)

    mout, sout, pout = pl.pallas_call(
        functools.partial(_fused_body, nblk_c=nblk_c, blk=blk, n_real=n),
        grid=(_NC, nblk_c),
        in_specs=[
            pl.BlockSpec((blk, _DIN), _rowmap),
            pl.BlockSpec((1, blk), lambda c, j: (0, jnp.minimum(
                c * nblk_c + j, nblk - 1))),
            pl.BlockSpec((_DIN, 2 * _NP * _DHID), lambda c, j: (0, 0)),
            pl.BlockSpec((1, 2 * _NP * _DHID), lambda c, j: (0, 0)),
            pl.BlockSpec((_NP * _DHID, _NP), lambda c, j: (0, 0)),
            pl.BlockSpec((1, _NP), lambda c, j: (0, 0)),
            pl.BlockSpec((_NP, _DHID, _DEMB), lambda c, j: (0, 0, 0)),
        ],
        out_specs=[
            pl.BlockSpec((1, 1, _NP), lambda c, j: (c, 0, 0)),
            pl.BlockSpec((1, _NG, _NP), lambda c, j: (c, 0, 0)),
            pl.BlockSpec((1, _NP, _NG, _DEMB), lambda c, j: (c, 0, 0, 0)),
        ],
        out_shape=[
            jax.ShapeDtypeStruct((_NC, 1, _NP), jnp.float32),
            jax.ShapeDtypeStruct((_NC, _NG, _NP), jnp.float32),
            jax.ShapeDtypeStruct((_NC, _NP, _NG, _DEMB), jnp.float32),
        ],
        scratch_shapes=[
            pltpu.VMEM((1, _NP), jnp.float32),
            pltpu.VMEM((_NG, _NP), jnp.float32),
            pltpu.VMEM((_NP, _NG, _DEMB), jnp.float32),
        ],
        compiler_params=pltpu.CompilerParams(
            dimension_semantics=("parallel", "arbitrary")),
        interpret=interpret,
    )(x, bf, w1, b1, wg2, bg2r, Wn2.astype(jnp.bfloat16))

    m = mout[:, 0, :]
    w = jnp.exp(m - jnp.max(m, axis=0, keepdims=True))
    s = jnp.sum(sout * w[:, None, :], axis=0)
    p = jnp.sum(pout * w[:, :, None, None], axis=0)
    st = jnp.transpose(s)[:, :, None]
    pooled = (p + st * bn2[:, None, :]) / (st + 1e-16)
    return jnp.transpose(pooled, (1, 0, 2)).reshape(_NG, _NP * _DEMB)


def kernel(x, edge_index, batch, n_nodes, Omegas, Phis, Lambdas,
           Omegas_norm, Phis_norm, Lambdas_norm,
           Wg1, bg1, Wg2, bg2, Wn1, bn1, Wn2, bn2):
    pooled = _pooled(x, batch, Wg1, bg1, Wg2, bg2, Wn1, bn1, Wn2, bn2)
    return jnp.concatenate([pooled, n_nodes, Omegas, Phis, Lambdas,
                            Omegas_norm, Phis_norm, Lambdas_norm], axis=1)


# speedup vs baseline: 1.0870x; 1.0870x over previous
import functools

import jax
import jax.numpy as jnp
from jax.experimental import pallas as pl
from jax.experimental.pallas import tpu as pltpu

_NG = 64
_DIN = 128
_DHID = 256
_DEMB = 124
_NP = 4
_BLK = 4096


def _fused_body(xref, bref, w1ref, b1ref, wg2ref, bg2ref, wn2ref, bn2ref,
                oref, mref, sref, pref, *, nblk, blk, n_real):
    i = pl.program_id(0)

    @pl.when(i == 0)
    def _init():
        mref[...] = jnp.full_like(mref, -1e30)
        sref[...] = jnp.zeros_like(sref)
        pref[...] = jnp.zeros_like(pref)

    rowi = jax.lax.broadcasted_iota(jnp.int32, (blk, 1), 0)
    valid = (rowi + i * blk) < n_real
    xb = jnp.where(valid, xref[...], 0.0).astype(jnp.bfloat16)
    h = jnp.dot(xb, w1ref[...], preferred_element_type=jnp.float32)
    hb = jnp.maximum(h + b1ref[...], 0.0).astype(jnp.bfloat16)
    hg = hb[:, : _NP * _DHID]
    g4 = jnp.dot(hg, wg2ref[...], preferred_element_type=jnp.float32)
    g4 = g4 + bg2ref[...]

    g4 = jnp.where(valid, g4, -1e30)

    mb = jnp.max(g4, axis=0, keepdims=True)
    mo = mref[...]
    mn = jnp.maximum(mo, mb)
    scale = jnp.exp(mo - mn)
    mref[...] = mn
    e4 = jnp.exp(g4 - mn)

    bb = bref[...].reshape(blk, 1)
    gid = jax.lax.broadcasted_iota(jnp.int32, (blk, _NG), 1).astype(jnp.float32)
    onehot = (bb == gid).astype(jnp.float32)
    snew = jax.lax.dot_general(onehot, e4, (((0,), (0,)), ((), ())),
                               preferred_element_type=jnp.float32)
    sref[...] = sref[...] * scale + snew

    for k in range(_NP):
        hk = hb[:, _NP * _DHID + k * _DHID: _NP * _DHID + (k + 1) * _DHID]
        ok = jnp.dot(hk, wn2ref[k], preferred_element_type=jnp.float32)
        wk = onehot * e4[:, k:k + 1]
        pk = jax.lax.dot_general(wk, ok, (((0,), (0,)), ((), ())),
                                 preferred_element_type=jnp.float32)
        pref[k] = pref[k] * scale[0, k] + pk

    @pl.when(i == nblk - 1)
    def _fin():
        s = sref[...]
        for k in range(_NP):
            oref[k] = (pref[k] + s[:, k:k + 1] * bn2ref[k]) / (
                s[:, k:k + 1] + 1e-16)


def _pooled(x, batch, Wg1, bg1, Wg2, bg2, Wn1, bn1, Wn2, bn2, interpret=False):
    n = x.shape[0]
    blk = _BLK
    nblk = -(-n // blk)
    bf = batch.astype(jnp.float32).reshape(1, n)

    w1 = jnp.concatenate(
        [jnp.transpose(Wg1, (1, 0, 2)).reshape(_DIN, _NP * _DHID),
         jnp.transpose(Wn1, (1, 0, 2)).reshape(_DIN, _NP * _DHID)],
        axis=1).astype(jnp.bfloat16)
    b1 = jnp.concatenate([bg1.reshape(1, -1), bn1.reshape(1, -1)], axis=1)
    ar = jnp.arange(_NP)
    wg2 = jnp.zeros((_NP, _DHID, _NP), jnp.float32)
    wg2 = wg2.at[ar, :, ar].set(Wg2[:, :, 0]).reshape(
        _NP * _DHID, _NP).astype(jnp.bfloat16)
    bg2r = bg2.reshape(1, _NP)
    bn2r = bn2.reshape(_NP, 1, _DEMB)

    out = pl.pallas_call(
        functools.partial(_fused_body, nblk=nblk, blk=blk, n_real=n),
        grid=(nblk,),
        in_specs=[
            pl.BlockSpec((blk, _DIN), lambda i: (i, 0)),
            pl.BlockSpec((1, blk), lambda i: (0, i)),
            pl.BlockSpec((_DIN, 2 * _NP * _DHID), lambda i: (0, 0)),
            pl.BlockSpec((1, 2 * _NP * _DHID), lambda i: (0, 0)),
            pl.BlockSpec((_NP * _DHID, _NP), lambda i: (0, 0)),
            pl.BlockSpec((1, _NP), lambda i: (0, 0)),
            pl.BlockSpec((_NP, _DHID, _DEMB), lambda i: (0, 0, 0)),
            pl.BlockSpec((_NP, 1, _DEMB), lambda i: (0, 0, 0)),
        ],
        out_specs=pl.BlockSpec((_NP, _NG, _DEMB), lambda i: (0, 0, 0)),
        out_shape=jax.ShapeDtypeStruct((_NP, _NG, _DEMB), jnp.float32),
        scratch_shapes=[
            pltpu.VMEM((1, _NP), jnp.float32),
            pltpu.VMEM((_NG, _NP), jnp.float32),
            pltpu.VMEM((_NP, _NG, _DEMB), jnp.float32),
        ],
        interpret=interpret,
    )(x, bf, w1, b1, wg2, bg2r, Wn2.astype(jnp.bfloat16), bn2r)
    return jnp.transpose(out, (1, 0, 2)).reshape(_NG, _NP * _DEMB)


def kernel(x, edge_index, batch, n_nodes, Omegas, Phis, Lambdas,
           Omegas_norm, Phis_norm, Lambdas_norm,
           Wg1, bg1, Wg2, bg2, Wn1, bn1, Wn2, bn2):
    pooled = _pooled(x, batch, Wg1, bg1, Wg2, bg2, Wn1, bn1, Wn2, bn2)
    return jnp.concatenate([pooled, n_nodes, Omegas, Phis, Lambdas,
                            Omegas_norm, Phis_norm, Lambdas_norm], axis=1)
